# baseline (speedup 1.0000x reference)
import functools

import jax
import jax.numpy as jnp
from jax import lax
from jax.experimental import pallas as pl
from jax.experimental.pallas import tpu as pltpu
from jax.experimental.pallas import tpu_sc as plsc

_PRIMES = (31, 43, 59, 61, 73, 97, 103, 113)
NH = 8
NBUC = 16384
D = 768
SH = 96
LN_EPS = 1e-12
L = 16
NC, NS = 2, 16
NW = NC * NS
C = 32
RPT = D // 128


def _make_sc_body(n_tok):
    tpw = n_tok // NW
    nchunks = tpw // C

    def body(ids_hbm, tbl_hbm, out_hbm, ids_v, idx_v, gbuf,
             sem_g0, sem_g1, sem_o0, sem_o1):
        cid = lax.axis_index("c")
        sid = lax.axis_index("s")
        wid = sid * NC + cid
        tok0 = wid * tpw
        sem_g = (sem_g0, sem_g1)
        sem_o = (sem_o0, sem_o1)

        pltpu.sync_copy(ids_hbm.at[pl.ds(tok0, tpw)], ids_v)

        def idx_ref(slot, i):
            return idx_v.at[pl.ds((slot * NH + i) * C, C)]

        def fire(ci, slot):
            base = ci * C

            def idx_body(j, c2):
                t1 = ids_v[pl.ds(base + j * L, L)] + 1
                for i in range(NH):
                    idx_v[pl.ds((slot * NH + i) * C + j * L, L)] = (
                        (t1 * _PRIMES[i]) & (NBUC - 1)) + i * NBUC
                return c2

            lax.fori_loop(0, C // L, idx_body, 0, unroll=True)
            for i in range(NH):
                pltpu.make_async_copy(
                    tbl_hbm.at[idx_ref(slot, i)],
                    gbuf.at[slot, pl.ds(i * C, C)], sem_g[slot]).start()

        def drain(slot):
            for i in range(NH):
                pltpu.make_async_copy(
                    tbl_hbm.at[idx_ref(slot, i)],
                    gbuf.at[slot, pl.ds(i * C, C)], sem_g[slot]).wait()

        def out_start(ci, slot):
            for i in range(NH):
                pltpu.make_async_copy(
                    gbuf.at[slot, pl.ds(i * C, C)],
                    out_hbm.at[pl.ds(tok0 + ci * C, C), pl.ds(i * SH, SH)],
                    sem_o[slot]).start()

        def out_wait(slot):
            for i in range(NH):
                pltpu.make_async_copy(
                    gbuf.at[slot, pl.ds(i * C, C)],
                    out_hbm.at[pl.ds(tok0, C), pl.ds(i * SH, SH)],
                    sem_o[slot]).wait()

        fire(0, 0)

        def pipe_body(i, carry):
            c1 = 2 * i + 1
            fire(c1, 1)
            drain(0)
            out_start(2 * i, 0)

            @pl.when(i < nchunks // 2 - 1)
            def _():
                out_wait(0)
                fire(c1 + 1, 0)

            drain(1)
            out_start(c1, 1)

            @pl.when(i < nchunks // 2 - 1)
            def _():
                out_wait(1)

            return carry

        lax.fori_loop(0, nchunks // 2, pipe_body, 0)
        out_wait(0)
        out_wait(1)

    return body


@functools.lru_cache(maxsize=None)
def _make_sc_call(n_tok):
    call = pl.kernel(
        _make_sc_body(n_tok),
        out_type=jax.ShapeDtypeStruct((n_tok, D), jnp.float32),
        mesh=plsc.VectorSubcoreMesh(
            core_axis_name="c", subcore_axis_name="s",
            num_cores=NC, num_subcores=NS),
        scratch_types=[
            pltpu.VMEM((n_tok // NW,), jnp.int32),
            pltpu.VMEM((2 * NH * C,), jnp.int32),
            pltpu.VMEM((2, NH * C, SH), jnp.float32),
        ] + [pltpu.SemaphoreType.DMA] * 4,
        compiler_params=pltpu.CompilerParams(use_tc_tiling_on_sc=False),
    )
    return jax.jit(call)


_TB = 512


def _post_body(g_ref, pos_ref, tt_ref, gam_ref, bet_ref, out_ref):
    x = g_ref[...].reshape(_TB, D)
    x = x + pos_ref[...] + tt_ref[0][None, :]
    mean = jnp.mean(x, axis=-1, keepdims=True)
    var = jnp.mean(jnp.square(x - mean), axis=-1, keepdims=True)
    y = (x - mean) * lax.rsqrt(var + LN_EPS)
    out_ref[...] = y * gam_ref[0][None, :] + bet_ref[0][None, :]


@functools.lru_cache(maxsize=None)
def _make_post_call(n_tok, s_len):
    nb = n_tok // s_len
    pb = s_len // _TB

    return jax.jit(pl.pallas_call(
        _post_body,
        grid=(pb, nb),
        in_specs=[
            pl.BlockSpec((_TB * RPT, 128), lambda p, b: (b * pb + p, 0)),
            pl.BlockSpec((_TB, D), lambda p, b: (p, 0)),
            pl.BlockSpec((1, D), lambda p, b: (0, 0)),
            pl.BlockSpec((1, D), lambda p, b: (0, 0)),
            pl.BlockSpec((1, D), lambda p, b: (0, 0)),
        ],
        out_specs=pl.BlockSpec((_TB, D), lambda p, b: (b * pb + p, 0)),
        out_shape=jax.ShapeDtypeStruct((n_tok, D), jnp.float32),
    ))


def kernel(text_t, mask, hash_tables, pos_table, tt_table, ln_gamma, ln_beta):
    del mask
    b, s = text_t.shape
    n_tok = b * s
    tpw = n_tok // NW
    assert n_tok % NW == 0 and tpw % (2 * C) == 0 and s % tpw == 0
    assert s % _TB == 0
    ids = text_t.reshape(n_tok)
    g = _make_sc_call(n_tok)(ids, hash_tables.reshape(NH * NBUC, SH))
    g = g.reshape(n_tok * RPT, 128)
    out = _make_post_call(n_tok, s)(
        g, pos_table, tt_table[:1], ln_gamma.reshape(1, D),
        ln_beta.reshape(1, D))
    return out.reshape(b, s, D)


# speedup vs baseline: 1.0907x; 1.0907x over previous
import functools

import jax
import jax.numpy as jnp
from jax import lax
from jax.experimental import pallas as pl
from jax.experimental.pallas import tpu as pltpu
from jax.experimental.pallas import tpu_sc as plsc

_PRIMES = (31, 43, 59, 61, 73, 97, 103, 113)
NH = 8
NBUC = 16384
D = 768
SH = 96
LN_EPS = 1e-12
L = 16
NC, NS = 2, 16
NW = NC * NS
C = 32
RPT = D // 128


def _make_sc_body(n_tok):
    tpw = n_tok // NW
    nchunks = tpw // C

    def body(ids_hbm, tbl_hbm, out_hbm, ids_v, idx_v, gbuf,
             sem_g0, sem_g1, sem_o0, sem_o1):
        cid = lax.axis_index("c")
        sid = lax.axis_index("s")
        wid = sid * NC + cid
        tok0 = wid * tpw
        sem_g = (sem_g0, sem_g1)
        sem_o = (sem_o0, sem_o1)

        pltpu.sync_copy(ids_hbm.at[pl.ds(tok0, tpw)], ids_v)

        def idx_ref(slot, i):
            return idx_v.at[pl.ds((slot * NH + i) * C, C)]

        def fire(ci, slot):
            base = ci * C

            def idx_body(j, c2):
                t1 = ids_v[pl.ds(base + j * L, L)] + 1
                for i in range(NH):
                    idx_v[pl.ds((slot * NH + i) * C + j * L, L)] = (
                        (t1 * _PRIMES[i]) & (NBUC - 1)) + i * NBUC
                return c2

            lax.fori_loop(0, C // L, idx_body, 0, unroll=True)
            for i in range(NH):
                pltpu.make_async_copy(
                    tbl_hbm.at[idx_ref(slot, i)],
                    gbuf.at[slot, pl.ds(i * C, C)], sem_g[slot]).start()

        def drain(slot):
            for i in range(NH):
                pltpu.make_async_copy(
                    tbl_hbm.at[idx_ref(slot, i)],
                    gbuf.at[slot, pl.ds(i * C, C)], sem_g[slot]).wait()

        def out_start(ci, slot):
            for i in range(NH):
                pltpu.make_async_copy(
                    gbuf.at[slot, pl.ds(i * C, C)],
                    out_hbm.at[pl.ds(tok0 + ci * C, C), pl.ds(i * SH, SH)],
                    sem_o[slot]).start()

        def out_wait(slot):
            for i in range(NH):
                pltpu.make_async_copy(
                    gbuf.at[slot, pl.ds(i * C, C)],
                    out_hbm.at[pl.ds(tok0, C), pl.ds(i * SH, SH)],
                    sem_o[slot]).wait()

        fire(0, 0)

        def pipe_body(i, carry):
            c1 = 2 * i + 1
            fire(c1, 1)
            drain(0)
            out_start(2 * i, 0)

            @pl.when(i < nchunks // 2 - 1)
            def _():
                out_wait(0)
                fire(c1 + 1, 0)

            drain(1)
            out_start(c1, 1)

            @pl.when(i < nchunks // 2 - 1)
            def _():
                out_wait(1)

            return carry

        lax.fori_loop(0, nchunks // 2, pipe_body, 0)
        out_wait(0)
        out_wait(1)

    return body


@functools.lru_cache(maxsize=None)
def _make_sc_call(n_tok):
    call = pl.kernel(
        _make_sc_body(n_tok),
        out_type=jax.ShapeDtypeStruct((n_tok, D), jnp.float32),
        mesh=plsc.VectorSubcoreMesh(
            core_axis_name="c", subcore_axis_name="s",
            num_cores=NC, num_subcores=NS),
        scratch_types=[
            pltpu.VMEM((n_tok // NW,), jnp.int32),
            pltpu.VMEM((2 * NH * C,), jnp.int32),
            pltpu.VMEM((2, NH * C, SH), jnp.float32),
        ] + [pltpu.SemaphoreType.DMA] * 4,
        compiler_params=pltpu.CompilerParams(use_tc_tiling_on_sc=False),
    )
    return jax.jit(call)


_TB = 512
_KS = 4


def _post_body(g_ref, pos_ref, tt_ref, gam_ref, bet_ref, out_ref):
    x = jnp.concatenate(
        [g_ref[pl.Slice(j, _TB, RPT), :] for j in range(RPT)], axis=-1)
    x = x + pos_ref[...] + tt_ref[0][None, :]
    mean = jnp.mean(x, axis=-1, keepdims=True)
    var = jnp.mean(jnp.square(x - mean), axis=-1, keepdims=True)
    y = (x - mean) * lax.rsqrt(var + LN_EPS)
    out_ref[...] = y * gam_ref[0][None, :] + bet_ref[0][None, :]


def _post_body_acc(prev_ref, g_ref, pos_ref, tt_ref, gam_ref, bet_ref,
                   out_ref):
    del prev_ref
    _post_body(g_ref, pos_ref, tt_ref, gam_ref, bet_ref, out_ref)


@functools.lru_cache(maxsize=None)
def _make_post_call(n_tok, s_len, p, first):
    nb = n_tok // s_len
    pbs = s_len // _KS // _TB
    pb = s_len // _TB

    in_specs = [
        pl.BlockSpec((_TB * RPT, 128), lambda q, b: (b * pbs + q, 0)),
        pl.BlockSpec((_TB, D), lambda q, b: (p * pbs + q, 0)),
        pl.BlockSpec((1, D), lambda q, b: (0, 0)),
        pl.BlockSpec((1, D), lambda q, b: (0, 0)),
        pl.BlockSpec((1, D), lambda q, b: (0, 0)),
    ]
    body = _post_body
    aliases = {}
    if not first:
        in_specs = [pl.BlockSpec((8, D), lambda q, b: (0, 0))] + in_specs
        body = _post_body_acc
        aliases = {0: 0}

    return jax.jit(pl.pallas_call(
        body,
        grid=(pbs, nb),
        in_specs=in_specs,
        out_specs=pl.BlockSpec(
            (_TB, D), lambda q, b: (b * pb + p * pbs + q, 0)),
        out_shape=jax.ShapeDtypeStruct((n_tok, D), jnp.float32),
        input_output_aliases=aliases,
    ))


def kernel(text_t, mask, hash_tables, pos_table, tt_table, ln_gamma, ln_beta):
    del mask
    b, s = text_t.shape
    n_tok = b * s
    sl = s // _KS
    n_st = b * sl
    tpw = n_st // NW
    assert n_st % NW == 0 and tpw % (2 * C) == 0
    assert sl % _TB == 0
    tbl = hash_tables.reshape(NH * NBUC, SH)
    gam = ln_gamma.reshape(1, D)
    bet = ln_beta.reshape(1, D)
    tt = tt_table[:1]
    sc = _make_sc_call(n_st)
    g2 = []
    for p in range(_KS):
        ids_p = text_t[:, p * sl:(p + 1) * sl].reshape(n_st)
        g2.append(sc(ids_p, tbl).reshape(n_st * RPT, 128))
    out = _make_post_call(n_tok, s, 0, True)(g2[0], pos_table, tt, gam, bet)
    for p in range(1, _KS):
        out = _make_post_call(n_tok, s, p, False)(
            out, g2[p], pos_table, tt, gam, bet)
    return out.reshape(b, s, D)


# speedup vs baseline: 1.1309x; 1.0368x over previous
import functools

import jax
import jax.numpy as jnp
from jax import lax
from jax.experimental import pallas as pl
from jax.experimental.pallas import tpu as pltpu
from jax.experimental.pallas import tpu_sc as plsc

_PRIMES = (31, 43, 59, 61, 73, 97, 103, 113)
NH = 8
NBUC = 16384
D = 768
SH = 96
LN_EPS = 1e-12
L = 16
NC, NS = 2, 16
NW = NC * NS
C = 32
RPT = D // 128


def _make_sc_body(n_tok):
    tpw = n_tok // NW
    nchunks = tpw // C

    def body(ids_hbm, tbl_hbm, out_hbm, ids_v, idx_v, gbuf,
             sem_g0, sem_g1, sem_o0, sem_o1):
        cid = lax.axis_index("c")
        sid = lax.axis_index("s")
        wid = sid * NC + cid
        tok0 = wid * tpw
        sem_g = (sem_g0, sem_g1)
        sem_o = (sem_o0, sem_o1)

        pltpu.sync_copy(ids_hbm.at[pl.ds(tok0, tpw)], ids_v)

        def idx_ref(slot, i):
            return idx_v.at[pl.ds((slot * NH + i) * C, C)]

        def fire(ci, slot):
            base = ci * C

            def idx_body(j, c2):
                t1 = ids_v[pl.ds(base + j * L, L)] + 1
                for i in range(NH):
                    idx_v[pl.ds((slot * NH + i) * C + j * L, L)] = (
                        (t1 * _PRIMES[i]) & (NBUC - 1)) + i * NBUC
                return c2

            lax.fori_loop(0, C // L, idx_body, 0, unroll=True)
            for i in range(NH):
                pltpu.make_async_copy(
                    tbl_hbm.at[idx_ref(slot, i)],
                    gbuf.at[slot, pl.ds(i * C, C)], sem_g[slot]).start()

        def drain(slot):
            for i in range(NH):
                pltpu.make_async_copy(
                    tbl_hbm.at[idx_ref(slot, i)],
                    gbuf.at[slot, pl.ds(i * C, C)], sem_g[slot]).wait()

        def out_start(ci, slot):
            for i in range(NH):
                pltpu.make_async_copy(
                    gbuf.at[slot, pl.ds(i * C, C)],
                    out_hbm.at[pl.ds(tok0 + ci * C, C), pl.ds(i * SH, SH)],
                    sem_o[slot]).start()

        def out_wait(slot):
            for i in range(NH):
                pltpu.make_async_copy(
                    gbuf.at[slot, pl.ds(i * C, C)],
                    out_hbm.at[pl.ds(tok0, C), pl.ds(i * SH, SH)],
                    sem_o[slot]).wait()

        fire(0, 0)

        def pipe_body(i, carry):
            c1 = 2 * i + 1
            fire(c1, 1)
            drain(0)
            out_start(2 * i, 0)

            @pl.when(i < nchunks // 2 - 1)
            def _():
                out_wait(0)
                fire(c1 + 1, 0)

            drain(1)
            out_start(c1, 1)

            @pl.when(i < nchunks // 2 - 1)
            def _():
                out_wait(1)

            return carry

        lax.fori_loop(0, nchunks // 2, pipe_body, 0)
        out_wait(0)
        out_wait(1)

    return body


@functools.lru_cache(maxsize=None)
def _make_sc_call(n_tok):
    call = pl.kernel(
        _make_sc_body(n_tok),
        out_type=jax.ShapeDtypeStruct((n_tok, D), jnp.float32),
        mesh=plsc.VectorSubcoreMesh(
            core_axis_name="c", subcore_axis_name="s",
            num_cores=NC, num_subcores=NS),
        scratch_types=[
            pltpu.VMEM((n_tok // NW,), jnp.int32),
            pltpu.VMEM((2 * NH * C,), jnp.int32),
            pltpu.VMEM((2, NH * C, SH), jnp.float32),
        ] + [pltpu.SemaphoreType.DMA] * 4,
        compiler_params=pltpu.CompilerParams(use_tc_tiling_on_sc=False),
    )
    return jax.jit(call)


_TB = 1024
_KS = 4


def _post_body(g_ref, pos_ref, tt_ref, gam_ref, bet_ref, out_ref):
    x = jnp.concatenate(
        [g_ref[pl.Slice(j, _TB, RPT), :] for j in range(RPT)], axis=-1)
    x = x + pos_ref[...] + tt_ref[0][None, :]
    mean = jnp.mean(x, axis=-1, keepdims=True)
    var = jnp.mean(jnp.square(x - mean), axis=-1, keepdims=True)
    y = (x - mean) * lax.rsqrt(var + LN_EPS)
    out_ref[...] = y * gam_ref[0][None, :] + bet_ref[0][None, :]


def _post_body_acc(prev_ref, g_ref, pos_ref, tt_ref, gam_ref, bet_ref,
                   out_ref):
    del prev_ref
    _post_body(g_ref, pos_ref, tt_ref, gam_ref, bet_ref, out_ref)


@functools.lru_cache(maxsize=None)
def _make_post_call(n_tok, s_len, p, first):
    nb = n_tok // s_len
    pbs = s_len // _KS // _TB
    pb = s_len // _TB

    in_specs = [
        pl.BlockSpec((_TB * RPT, 128), lambda q, b: (b * pbs + q, 0)),
        pl.BlockSpec((_TB, D), lambda q, b: (p * pbs + q, 0)),
        pl.BlockSpec((1, D), lambda q, b: (0, 0)),
        pl.BlockSpec((1, D), lambda q, b: (0, 0)),
        pl.BlockSpec((1, D), lambda q, b: (0, 0)),
    ]
    body = _post_body
    aliases = {}
    if not first:
        in_specs = [pl.BlockSpec((8, D), lambda q, b: (0, 0))] + in_specs
        body = _post_body_acc
        aliases = {0: 0}

    return jax.jit(pl.pallas_call(
        body,
        grid=(pbs, nb),
        in_specs=in_specs,
        out_specs=pl.BlockSpec(
            (_TB, D), lambda q, b: (b * pb + p * pbs + q, 0)),
        out_shape=jax.ShapeDtypeStruct((n_tok, D), jnp.float32),
        input_output_aliases=aliases,
    ))


def kernel(text_t, mask, hash_tables, pos_table, tt_table, ln_gamma, ln_beta):
    del mask
    b, s = text_t.shape
    n_tok = b * s
    sl = s // _KS
    n_st = b * sl
    tpw = n_st // NW
    assert n_st % NW == 0 and tpw % (2 * C) == 0
    assert sl % _TB == 0
    tbl = hash_tables.reshape(NH * NBUC, SH)
    gam = ln_gamma.reshape(1, D)
    bet = ln_beta.reshape(1, D)
    tt = tt_table[:1]
    sc = _make_sc_call(n_st)
    g2 = []
    for p in range(_KS):
        ids_p = text_t[:, p * sl:(p + 1) * sl].reshape(n_st)
        g2.append(sc(ids_p, tbl).reshape(n_st * RPT, 128))
    out = _make_post_call(n_tok, s, 0, True)(g2[0], pos_table, tt, gam, bet)
    for p in range(1, _KS):
        out = _make_post_call(n_tok, s, p, False)(
            out, g2[p], pos_table, tt, gam, bet)
    return out.reshape(b, s, D)


# speedup vs baseline: 1.1517x; 1.0184x over previous
import functools

import jax
import jax.numpy as jnp
from jax import lax
from jax.experimental import pallas as pl
from jax.experimental.pallas import tpu as pltpu
from jax.experimental.pallas import tpu_sc as plsc

_PRIMES = (31, 43, 59, 61, 73, 97, 103, 113)
NH = 8
NBUC = 16384
D = 768
SH = 96
LN_EPS = 1e-12
L = 16
NC, NS = 2, 16
NW = NC * NS
C = 32
RPT = D // 128


def _make_sc_body(n_tok):
    tpw = n_tok // NW
    nchunks = tpw // C

    def body(ids_hbm, tbl_hbm, out_hbm, ids_v, idx_v, gbuf,
             sem_g0, sem_g1, sem_o0, sem_o1):
        cid = lax.axis_index("c")
        sid = lax.axis_index("s")
        wid = sid * NC + cid
        tok0 = wid * tpw
        sem_g = (sem_g0, sem_g1)
        sem_o = (sem_o0, sem_o1)

        pltpu.sync_copy(ids_hbm.at[pl.ds(tok0, tpw)], ids_v)

        def idx_ref(slot, i):
            return idx_v.at[pl.ds((slot * NH + i) * C, C)]

        def fire(ci, slot):
            base = ci * C

            def idx_body(j, c2):
                t1 = ids_v[pl.ds(base + j * L, L)] + 1
                for i in range(NH):
                    idx_v[pl.ds((slot * NH + i) * C + j * L, L)] = (
                        (t1 * _PRIMES[i]) & (NBUC - 1)) + i * NBUC
                return c2

            lax.fori_loop(0, C // L, idx_body, 0, unroll=True)
            for i in range(NH):
                pltpu.make_async_copy(
                    tbl_hbm.at[idx_ref(slot, i)],
                    gbuf.at[slot, pl.ds(i * C, C)], sem_g[slot]).start()

        def drain(slot):
            for i in range(NH):
                pltpu.make_async_copy(
                    tbl_hbm.at[idx_ref(slot, i)],
                    gbuf.at[slot, pl.ds(i * C, C)], sem_g[slot]).wait()

        def out_start(ci, slot):
            for i in range(NH):
                pltpu.make_async_copy(
                    gbuf.at[slot, pl.ds(i * C, C)],
                    out_hbm.at[pl.ds(tok0 + ci * C, C), pl.ds(i * SH, SH)],
                    sem_o[slot]).start()

        def out_wait(slot):
            for i in range(NH):
                pltpu.make_async_copy(
                    gbuf.at[slot, pl.ds(i * C, C)],
                    out_hbm.at[pl.ds(tok0, C), pl.ds(i * SH, SH)],
                    sem_o[slot]).wait()

        fire(0, 0)

        def pipe_body(i, carry):
            c1 = 2 * i + 1
            fire(c1, 1)
            drain(0)
            out_start(2 * i, 0)

            @pl.when(i < nchunks // 2 - 1)
            def _():
                out_wait(0)
                fire(c1 + 1, 0)

            drain(1)
            out_start(c1, 1)

            @pl.when(i < nchunks // 2 - 1)
            def _():
                out_wait(1)

            return carry

        lax.fori_loop(0, nchunks // 2, pipe_body, 0)
        out_wait(0)
        out_wait(1)

    return body


@functools.lru_cache(maxsize=None)
def _make_sc_call(n_tok):
    call = pl.kernel(
        _make_sc_body(n_tok),
        out_type=jax.ShapeDtypeStruct((n_tok, D), jnp.float32),
        mesh=plsc.VectorSubcoreMesh(
            core_axis_name="c", subcore_axis_name="s",
            num_cores=NC, num_subcores=NS),
        scratch_types=[
            pltpu.VMEM((n_tok // NW,), jnp.int32),
            pltpu.VMEM((2 * NH * C,), jnp.int32),
            pltpu.VMEM((2, NH * C, SH), jnp.float32),
        ] + [pltpu.SemaphoreType.DMA] * 4,
        compiler_params=pltpu.CompilerParams(use_tc_tiling_on_sc=False),
    )
    return jax.jit(call)


_TB = 2048
_KS = 4


def _post_body(g_ref, pos_ref, tt_ref, gam_ref, bet_ref, out_ref):
    x = jnp.concatenate(
        [g_ref[pl.Slice(j, _TB, RPT), :] for j in range(RPT)], axis=-1)
    x = x + pos_ref[...] + tt_ref[0][None, :]
    mean = jnp.mean(x, axis=-1, keepdims=True)
    var = jnp.mean(jnp.square(x - mean), axis=-1, keepdims=True)
    y = (x - mean) * lax.rsqrt(var + LN_EPS)
    out_ref[...] = y * gam_ref[0][None, :] + bet_ref[0][None, :]


def _post_body_acc(prev_ref, g_ref, pos_ref, tt_ref, gam_ref, bet_ref,
                   out_ref):
    del prev_ref
    _post_body(g_ref, pos_ref, tt_ref, gam_ref, bet_ref, out_ref)


@functools.lru_cache(maxsize=None)
def _make_post_call(n_tok, s_len, p, first):
    nb = n_tok // s_len
    pbs = s_len // _KS // _TB
    pb = s_len // _TB

    in_specs = [
        pl.BlockSpec((_TB * RPT, 128), lambda q, b: (b * pbs + q, 0)),
        pl.BlockSpec((_TB, D), lambda q, b: (p * pbs + q, 0)),
        pl.BlockSpec((1, D), lambda q, b: (0, 0)),
        pl.BlockSpec((1, D), lambda q, b: (0, 0)),
        pl.BlockSpec((1, D), lambda q, b: (0, 0)),
    ]
    body = _post_body
    aliases = {}
    if not first:
        in_specs = [pl.BlockSpec((8, D), lambda q, b: (0, 0))] + in_specs
        body = _post_body_acc
        aliases = {0: 0}

    return jax.jit(pl.pallas_call(
        body,
        grid=(pbs, nb),
        in_specs=in_specs,
        out_specs=pl.BlockSpec(
            (_TB, D), lambda q, b: (b * pb + p * pbs + q, 0)),
        out_shape=jax.ShapeDtypeStruct((n_tok, D), jnp.float32),
        input_output_aliases=aliases,
    ))


def kernel(text_t, mask, hash_tables, pos_table, tt_table, ln_gamma, ln_beta):
    del mask
    b, s = text_t.shape
    n_tok = b * s
    sl = s // _KS
    n_st = b * sl
    tpw = n_st // NW
    assert n_st % NW == 0 and tpw % (2 * C) == 0
    assert sl % _TB == 0
    tbl = hash_tables.reshape(NH * NBUC, SH)
    gam = ln_gamma.reshape(1, D)
    bet = ln_beta.reshape(1, D)
    tt = tt_table[:1]
    sc = _make_sc_call(n_st)
    g2 = []
    for p in range(_KS):
        ids_p = text_t[:, p * sl:(p + 1) * sl].reshape(n_st)
        g2.append(sc(ids_p, tbl).reshape(n_st * RPT, 128))
    out = _make_post_call(n_tok, s, 0, True)(g2[0], pos_table, tt, gam, bet)
    for p in range(1, _KS):
        out = _make_post_call(n_tok, s, p, False)(
            out, g2[p], pos_table, tt, gam, bet)
    return out.reshape(b, s, D)


# speedup vs baseline: 1.1570x; 1.0046x over previous
import functools

import jax
import jax.numpy as jnp
from jax import lax
from jax.experimental import pallas as pl
from jax.experimental.pallas import tpu as pltpu
from jax.experimental.pallas import tpu_sc as plsc

_PRIMES = (31, 43, 59, 61, 73, 97, 103, 113)
NH = 8
NBUC = 16384
D = 768
SH = 96
LN_EPS = 1e-12
L = 16
NC, NS = 2, 16
NW = NC * NS
C = 32
RPT = D // 128


def _make_sc_body(n_tok):
    tpw = n_tok // NW
    nchunks = tpw // C

    def body(ids_hbm, tbl_hbm, out_hbm, ids_v, idx_v, gbuf,
             sem_g0, sem_g1, sem_o0, sem_o1):
        cid = lax.axis_index("c")
        sid = lax.axis_index("s")
        wid = sid * NC + cid
        tok0 = wid * tpw
        sem_g = (sem_g0, sem_g1)
        sem_o = (sem_o0, sem_o1)

        pltpu.sync_copy(ids_hbm.at[pl.ds(tok0, tpw)], ids_v)

        def idx_ref(slot, i):
            return idx_v.at[pl.ds((slot * NH + i) * C, C)]

        def fire(ci, slot):
            base = ci * C

            def idx_body(j, c2):
                t1 = ids_v[pl.ds(base + j * L, L)] + 1
                for i in range(NH):
                    idx_v[pl.ds((slot * NH + i) * C + j * L, L)] = (
                        (t1 * _PRIMES[i]) & (NBUC - 1)) + i * NBUC
                return c2

            lax.fori_loop(0, C // L, idx_body, 0, unroll=True)
            for i in range(NH):
                pltpu.make_async_copy(
                    tbl_hbm.at[idx_ref(slot, i)],
                    gbuf.at[slot, pl.ds(i * C, C)], sem_g[slot]).start()

        def drain(slot):
            for i in range(NH):
                pltpu.make_async_copy(
                    tbl_hbm.at[idx_ref(slot, i)],
                    gbuf.at[slot, pl.ds(i * C, C)], sem_g[slot]).wait()

        def out_start(ci, slot):
            for i in range(NH):
                pltpu.make_async_copy(
                    gbuf.at[slot, pl.ds(i * C, C)],
                    out_hbm.at[pl.ds(tok0 + ci * C, C), pl.ds(i * SH, SH)],
                    sem_o[slot]).start()

        def out_wait(slot):
            for i in range(NH):
                pltpu.make_async_copy(
                    gbuf.at[slot, pl.ds(i * C, C)],
                    out_hbm.at[pl.ds(tok0, C), pl.ds(i * SH, SH)],
                    sem_o[slot]).wait()

        fire(0, 0)

        def pipe_body(i, carry):
            c1 = 2 * i + 1
            fire(c1, 1)
            drain(0)
            out_start(2 * i, 0)

            @pl.when(i < nchunks // 2 - 1)
            def _():
                out_wait(0)
                fire(c1 + 1, 0)

            drain(1)
            out_start(c1, 1)

            @pl.when(i < nchunks // 2 - 1)
            def _():
                out_wait(1)

            return carry

        lax.fori_loop(0, nchunks // 2, pipe_body, 0)
        out_wait(0)
        out_wait(1)

    return body


@functools.lru_cache(maxsize=None)
def _make_sc_call(n_tok):
    call = pl.kernel(
        _make_sc_body(n_tok),
        out_type=jax.ShapeDtypeStruct((n_tok, D), jnp.float32),
        mesh=plsc.VectorSubcoreMesh(
            core_axis_name="c", subcore_axis_name="s",
            num_cores=NC, num_subcores=NS),
        scratch_types=[
            pltpu.VMEM((n_tok // NW,), jnp.int32),
            pltpu.VMEM((2 * NH * C,), jnp.int32),
            pltpu.VMEM((2, NH * C, SH), jnp.float32),
        ] + [pltpu.SemaphoreType.DMA] * 4,
        compiler_params=pltpu.CompilerParams(use_tc_tiling_on_sc=False),
    )
    return jax.jit(call)


_TB = 2048
_KS = 2


def _post_body(g_ref, pos_ref, tt_ref, gam_ref, bet_ref, out_ref):
    x = jnp.concatenate(
        [g_ref[pl.Slice(j, _TB, RPT), :] for j in range(RPT)], axis=-1)
    x = x + pos_ref[...] + tt_ref[0][None, :]
    mean = jnp.mean(x, axis=-1, keepdims=True)
    var = jnp.mean(jnp.square(x - mean), axis=-1, keepdims=True)
    y = (x - mean) * lax.rsqrt(var + LN_EPS)
    out_ref[...] = y * gam_ref[0][None, :] + bet_ref[0][None, :]


def _post_body_acc(prev_ref, g_ref, pos_ref, tt_ref, gam_ref, bet_ref,
                   out_ref):
    del prev_ref
    _post_body(g_ref, pos_ref, tt_ref, gam_ref, bet_ref, out_ref)


@functools.lru_cache(maxsize=None)
def _make_post_call(n_tok, s_len, p, first):
    nb = n_tok // s_len
    pbs = s_len // _KS // _TB
    pb = s_len // _TB

    in_specs = [
        pl.BlockSpec((_TB * RPT, 128), lambda q, b: (b * pbs + q, 0)),
        pl.BlockSpec((_TB, D), lambda q, b: (p * pbs + q, 0)),
        pl.BlockSpec((1, D), lambda q, b: (0, 0)),
        pl.BlockSpec((1, D), lambda q, b: (0, 0)),
        pl.BlockSpec((1, D), lambda q, b: (0, 0)),
    ]
    body = _post_body
    aliases = {}
    if not first:
        in_specs = [pl.BlockSpec((8, D), lambda q, b: (0, 0))] + in_specs
        body = _post_body_acc
        aliases = {0: 0}

    return jax.jit(pl.pallas_call(
        body,
        grid=(pbs, nb),
        in_specs=in_specs,
        out_specs=pl.BlockSpec(
            (_TB, D), lambda q, b: (b * pb + p * pbs + q, 0)),
        out_shape=jax.ShapeDtypeStruct((n_tok, D), jnp.float32),
        input_output_aliases=aliases,
    ))


def kernel(text_t, mask, hash_tables, pos_table, tt_table, ln_gamma, ln_beta):
    del mask
    b, s = text_t.shape
    n_tok = b * s
    sl = s // _KS
    n_st = b * sl
    tpw = n_st // NW
    assert n_st % NW == 0 and tpw % (2 * C) == 0
    assert sl % _TB == 0
    tbl = hash_tables.reshape(NH * NBUC, SH)
    gam = ln_gamma.reshape(1, D)
    bet = ln_beta.reshape(1, D)
    tt = tt_table[:1]
    sc = _make_sc_call(n_st)
    g2 = []
    for p in range(_KS):
        ids_p = text_t[:, p * sl:(p + 1) * sl].reshape(n_st)
        g2.append(sc(ids_p, tbl).reshape(n_st * RPT, 128))
    out = _make_post_call(n_tok, s, 0, True)(g2[0], pos_table, tt, gam, bet)
    for p in range(1, _KS):
        out = _make_post_call(n_tok, s, p, False)(
            out, g2[p], pos_table, tt, gam, bet)
    return out.reshape(b, s, D)
